# Initial kernel scaffold; baseline (speedup 1.0000x reference)
import jax
import jax.numpy as jnp
from jax.experimental import pallas as pl


def kernel(node_idx, attenuation):
    raise NotImplementedError("write your pallas kernel here")


# speedup vs baseline: 1.0600x; 1.0600x over previous
import functools

import jax
import jax.numpy as jnp
from jax import lax
from jax.experimental import pallas as pl
from jax.experimental.pallas import tpu as pltpu
from jax.experimental.pallas import tpu_sc as plsc

EMB_SIZE = 256
NUM_NODES = 100000
TABLE_SEED = 42
BATCH = 50000

_NC = 2
_NS = 16
_NW = _NC * _NS
_L = 16

_C = 128
_NCH = -(-BATCH // _C)
_LAST_BASE = BATCH - _C
_K = -(-_NCH // _NW)

def _node_embs():
    return jax.random.normal(
        jax.random.key(TABLE_SEED), (NUM_NODES, EMB_SIZE), dtype=jnp.float32
    ) / jnp.sqrt(jnp.float32(EMB_SIZE))


@functools.partial(
    pl.kernel,
    out_type=jax.ShapeDtypeStruct((BATCH, EMB_SIZE), jnp.float32),
    mesh=plsc.VectorSubcoreMesh(core_axis_name="c", subcore_axis_name="s"),
    scratch_types=[
        pltpu.VMEM((_C,), jnp.int32),
        pltpu.VMEM((_C,), jnp.float32),
        pltpu.VMEM((_C, EMB_SIZE), jnp.float32),
        pltpu.SemaphoreType.DMA,
        pltpu.SemaphoreType.DMA,
    ],
)
def _gather_scale(emb_hbm, idx_hbm, att_hbm, out_hbm,
                  idx_v, att_v, rows_v, sem_r, sem_a):
    wid = lax.axis_index("s") * _NC + lax.axis_index("c")

    def chunk_body(k, carry):
        cid = wid + k * _NW

        @pl.when(cid < _NCH)
        def _():
            base = jnp.minimum(cid * _C, _LAST_BASE)
            pltpu.sync_copy(idx_hbm.at[pl.ds(base, _C)], idx_v)
            row_cp = pltpu.async_copy(emb_hbm.at[idx_v], rows_v, sem_r)
            att_cp = pltpu.async_copy(att_hbm.at[idx_v], att_v, sem_a)
            att_cp.wait()
            row_cp.wait()

            def grp_body(g, c2):
                att16 = att_v[pl.ds(g * _L, _L)]
                base_r = g * _L
                for l in range(_L):
                    a = jnp.broadcast_to(att16[l], (_L,))
                    for j in range(EMB_SIZE // _L):
                        sl = pl.ds(j * _L, _L)
                        rows_v[base_r + l, sl] = rows_v[base_r + l, sl] * a
                return c2

            lax.fori_loop(0, _C // _L, grp_body, 0)
            pltpu.sync_copy(rows_v, out_hbm.at[pl.ds(base, _C)])

        return carry

    lax.fori_loop(0, _K, chunk_body, 0)


def kernel(node_idx, attenuation):
    return _gather_scale(_node_embs(), node_idx, attenuation)
